# baseline (speedup 1.0000x reference)
import functools
import jax
import jax.numpy as jnp
from jax import lax
from jax.experimental import pallas as pl
from jax.experimental.pallas import tpu as pltpu
from jax.experimental.pallas import tpu_sc as plsc

N = 10000
NP = 10240
E = 320000
H = 128
HH = 64
L = 3
T = 32768
FC = 1000
OD = 474

NC, NS, LN = 2, 16, 16
NW = NC * NS

_SC_PARAMS = pltpu.CompilerParams(use_tc_tiling_on_sc=False)
_mesh = plsc.VectorSubcoreMesh(core_axis_name="c", subcore_axis_name="s")


def _make_gather(V, B, G):
  assert B % NW == 0
  b_per_w = B // NW
  assert b_per_w % G == 0 and G % 8 == 0

  @functools.partial(
      pl.kernel, mesh=_mesh, compiler_params=_SC_PARAMS,
      out_type=jax.ShapeDtypeStruct((B, H), jnp.float32),
      scratch_types=[
          pltpu.VMEM((G,), jnp.int32),
          pltpu.VMEM((G, H), jnp.float32),
          pltpu.SemaphoreType.DMA,
      ],
  )
  def k(table_h, idx_h, out_h, idx_v, rows_v, sem):
    wid = lax.axis_index("s") * NC + lax.axis_index("c")

    @pl.loop(0, b_per_w, step=G)
    def _(i):
      base = wid * b_per_w + i
      pltpu.sync_copy(idx_h.at[pl.ds(base, G)], idx_v)
      pltpu.async_copy(table_h.at[idx_v], rows_v, sem).wait()
      pltpu.sync_copy(rows_v, out_h.at[pl.ds(base, G)])

  return k


EB = 80
E_PER_S = E // NS
NBLK = E_PER_S // EB
ROWS_PER_S = NP // NS


def _make_edge(write_eij):
  outs = [jax.ShapeDtypeStruct((2, NP, H), jnp.float32)]
  if write_eij:
    outs.append(jax.ShapeDtypeStruct((2, E, HH), jnp.float32))

  @functools.partial(
      pl.kernel, mesh=_mesh, compiler_params=_SC_PARAMS,
      out_type=outs,
      scratch_types=[
          pltpu.VMEM((2, EB), jnp.int32),
          pltpu.VMEM((2, EB), jnp.int32),
          pltpu.VMEM((EB, H), jnp.float32),
          pltpu.VMEM((EB, HH), jnp.float32),
          pltpu.VMEM((EB, H), jnp.float32),
          pltpu.VMEM((EB, HH), jnp.float32),
          pltpu.VMEM((EB, HH), jnp.float32),
          pltpu.VMEM((EB, H), jnp.float32),
          pltpu.VMEM_SHARED((NP, H), jnp.float32),
          pltpu.SemaphoreType.DMA,
          pltpu.SemaphoreType.DMA,
      ],
  )
  def k(src_h, dst_h, db_h, eh_h, ce_h, *rest):
    if write_eij:
      nd_h, eij_h = rest[:2]
      scratch = rest[2:]
    else:
      nd_h = rest[0]
      scratch = rest[1:]
    (idxa, idxb, gdba, g2a, gdbb, g2b, cev, scomb, acc,
     sem_a, sem_b) = scratch
    c = lax.axis_index("c")
    sid = lax.axis_index("s")

    @pl.loop(0, EB)
    def _(r):
      @pl.loop(0, H, step=LN)
      def _(j):
        scomb[r, pl.ds(j, LN)] = jnp.zeros((LN,), jnp.float32)

    @pl.loop(0, ROWS_PER_S, step=EB)
    def _(r):
      pltpu.sync_copy(scomb, acc.at[pl.ds(sid * ROWS_PER_S + r, EB)])

    plsc.subcore_barrier()

    def fetch_idx(blk, idx2):
      pltpu.sync_copy(src_h.at[sid, blk], idx2.at[0])
      pltpu.sync_copy(dst_h.at[sid, blk], idx2.at[1])

    def issue(idx2, gdb, g2, sem):
      pltpu.async_copy(db_h.at[c].at[idx2.at[0]], gdb, sem)
      pltpu.async_copy(eh_h.at[c].at[idx2.at[1]], g2, sem)

    def wait_set(idx2, gdb, g2, sem):
      pltpu.make_async_copy(db_h.at[c].at[idx2.at[0]], gdb, sem).wait()
      pltpu.make_async_copy(eh_h.at[c].at[idx2.at[1]], g2, sem).wait()

    def phase(blk, idx2, gdb, g2, sem):
      pltpu.sync_copy(
          ce_h.at[c, pl.ds(sid * E_PER_S + blk * EB, EB)], cev)
      wait_set(idx2, gdb, g2, sem)

      @pl.loop(0, EB)
      def _(r):
        @pl.loop(0, HH, step=LN)
        def _(j):
          eij = (gdb[r, pl.ds(j, LN)] + g2[r, pl.ds(j, LN)]
                 + cev[r, pl.ds(j, LN)])
          sg = 1.0 / (1.0 + jnp.exp(-eij))
          sgb = sg * gdb[r, pl.ds(HH + j, LN)]
          gdb[r, pl.ds(j, LN)] = eij
          scomb[r, pl.ds(HH + j, LN)] = sg
          scomb[r, pl.ds(j, LN)] = sgb

      if write_eij:
        pltpu.sync_copy(
            gdb.at[:, pl.ds(0, HH)],
            eij_h.at[c, pl.ds(sid * E_PER_S + blk * EB, EB)])
      pltpu.sync_copy(scomb, acc.at[idx2.at[1]], add=True)

      @pl.when(blk + 2 < NBLK)
      def _():
        fetch_idx(blk + 2, idx2)
        issue(idx2, gdb, g2, sem)

    fetch_idx(0, idxa)
    issue(idxa, gdba, g2a, sem_a)
    fetch_idx(1, idxb)
    issue(idxb, gdbb, g2b, sem_b)

    @pl.loop(0, NBLK, step=2)
    def _(i):
      phase(i, idxa, gdba, g2a, sem_a)
      phase(i + 1, idxb, gdbb, g2b, sem_b)

    plsc.subcore_barrier()
    rb = sid * ROWS_PER_S
    pltpu.sync_copy(acc.at[pl.ds(rb, ROWS_PER_S)],
                    nd_h.at[c, pl.ds(rb, ROWS_PER_S)])

  return k


def _dot(a, b):
  return lax.dot_general(a, b, (((1,), (0,)), ((), ())),
                         preferred_element_type=jnp.float32)


def _node_mm_body(h_ref, aw, bw, dw, ew, ab, bb, db, eb,
                  ah_o, db_o, eh_o):
  h = h_ref[...]
  ah_o[...] = _dot(h, aw[...]) + ab[...]
  dh = _dot(h, dw[...]) + db[...]
  eh = _dot(h, ew[...]) + eb[...]
  bh = _dot(h, bw[...]) + bb[...]
  db_o[0] = jnp.concatenate([dh[:, :HH], bh[:, :HH]], axis=1)
  db_o[1] = jnp.concatenate([dh[:, HH:], bh[:, HH:]], axis=1)
  eh_o[0] = eh[:, :HH]
  eh_o[1] = eh[:, HH:]


def _node_mm(h, aw, bw, dw, ew, ab, bb, db, eb):
  BR = 2560
  g = NP // BR
  wspec = pl.BlockSpec((H, H), lambda i: (0, 0))
  bspec = pl.BlockSpec((1, H), lambda i: (0, 0))
  return pl.pallas_call(
      _node_mm_body,
      grid=(g,),
      in_specs=[pl.BlockSpec((BR, H), lambda i: (i, 0))] + [wspec] * 4
      + [bspec] * 4,
      out_specs=[
          pl.BlockSpec((BR, H), lambda i: (i, 0)),
          pl.BlockSpec((2, BR, H), lambda i: (0, i, 0)),
          pl.BlockSpec((2, BR, HH), lambda i: (0, i, 0)),
      ],
      out_shape=[
          jax.ShapeDtypeStruct((NP, H), jnp.float32),
          jax.ShapeDtypeStruct((2, NP, H), jnp.float32),
          jax.ShapeDtypeStruct((2, NP, HH), jnp.float32),
      ],
  )(h, aw, bw, dw, ew, ab, bb, db, eb)


def _h_update_body(ah_ref, nd_ref, nn_ref, hin_ref, g_ref, b_ref,
                   out_ref):
  num = jnp.concatenate([nd_ref[0][:, :HH], nd_ref[1][:, :HH]], axis=1)
  den = jnp.concatenate([nd_ref[0][:, HH:], nd_ref[1][:, HH:]], axis=1)
  t = (ah_ref[...] + num / (den + 1e-6)) * nn_ref[...]
  rid = lax.broadcasted_iota(jnp.int32, (NP, H), 0)
  mask = rid < N
  tm = jnp.where(mask, t, 0.0)
  m = jnp.sum(tm, axis=0, keepdims=True) / N
  v = jnp.sum(tm * tm, axis=0, keepdims=True) / N - m * m
  y = (t - m) / jnp.sqrt(v + 1e-5) * g_ref[...] + b_ref[...]
  out_ref[...] = hin_ref[...] + jnp.maximum(y, 0.0)


def _h_update(ah, nd, nn, hin, g, b):
  return pl.pallas_call(
      _h_update_body,
      grid=(1,),
      in_specs=[
          pl.BlockSpec((NP, H), lambda i: (0, 0)),
          pl.BlockSpec((2, NP, H), lambda i: (0, 0, 0)),
          pl.BlockSpec((NP, 1), lambda i: (0, 0)),
          pl.BlockSpec((NP, H), lambda i: (0, 0)),
          pl.BlockSpec((1, H), lambda i: (0, 0)),
          pl.BlockSpec((1, H), lambda i: (0, 0)),
      ],
      out_specs=pl.BlockSpec((NP, H), lambda i: (0, 0)),
      out_shape=jax.ShapeDtypeStruct((NP, H), jnp.float32),
  )(ah, nd, nn, hin, g, b)


def _estats_body(eij_ref, ne_ref, out_ref):
  i = pl.program_id(0)

  @pl.when(i == 0)
  def _():
    out_ref[...] = jnp.zeros_like(out_ref)

  y = jnp.concatenate([eij_ref[0], eij_ref[1]], axis=1) * ne_ref[...]
  s = jnp.sum(y, axis=0, keepdims=True)
  s2 = jnp.sum(y * y, axis=0, keepdims=True)
  out_ref[0:1, :] += s
  out_ref[1:2, :] += s2


def _estats(eij, ne):
  BR = 8000
  g = E // BR
  return pl.pallas_call(
      _estats_body,
      grid=(g,),
      in_specs=[
          pl.BlockSpec((2, BR, HH), lambda i: (0, i, 0)),
          pl.BlockSpec((BR, 1), lambda i: (i, 0)),
      ],
      out_specs=pl.BlockSpec((8, H), lambda i: (0, 0)),
      out_shape=jax.ShapeDtypeStruct((8, H), jnp.float32),
  )(eij, ne)


def _eupdate_ce_body(write_e, eij_ref, ein_ref, ne_ref, st_ref, g_ref, b_ref,
                     cw_ref, cb_ref, *outs):
  y = jnp.concatenate([eij_ref[0], eij_ref[1]], axis=1) * ne_ref[...]
  m = st_ref[0:1, :] / E
  v = st_ref[1:2, :] / E - m * m
  yn = (y - m) / jnp.sqrt(v + 1e-5) * g_ref[...] + b_ref[...]
  e_new = ein_ref[...] + jnp.maximum(yn, 0.0)
  ce = _dot(e_new, cw_ref[...]) + cb_ref[...]
  if write_e:
    ce_o, e_o = outs
    e_o[...] = e_new
  else:
    (ce_o,) = outs
  ce_o[0] = ce[:, :HH]
  ce_o[1] = ce[:, HH:]


def _eupdate_ce(eij, ein, ne, st, g, b, cw, cb, write_e):
  BR = 2000
  grid = E // BR
  out_specs = [pl.BlockSpec((2, BR, HH), lambda i: (0, i, 0))]
  out_shape = [jax.ShapeDtypeStruct((2, E, HH), jnp.float32)]
  if write_e:
    out_specs.append(pl.BlockSpec((BR, H), lambda i: (i, 0)))
    out_shape.append(jax.ShapeDtypeStruct((E, H), jnp.float32))
  return pl.pallas_call(
      functools.partial(_eupdate_ce_body, write_e),
      grid=(grid,),
      in_specs=[
          pl.BlockSpec((2, BR, HH), lambda i: (0, i, 0)),
          pl.BlockSpec((BR, H), lambda i: (i, 0)),
          pl.BlockSpec((BR, 1), lambda i: (i, 0)),
          pl.BlockSpec((8, H), lambda i: (0, 0)),
          pl.BlockSpec((1, H), lambda i: (0, 0)),
          pl.BlockSpec((1, H), lambda i: (0, 0)),
          pl.BlockSpec((H, H), lambda i: (0, 0)),
          pl.BlockSpec((1, H), lambda i: (0, 0)),
      ],
      out_specs=out_specs,
      out_shape=out_shape,
  )(eij, ein, ne, st, g, b, cw, cb)


def _ce0_body(e_ref, cw_ref, cb_ref, ce_o):
  ce = _dot(e_ref[...], cw_ref[...]) + cb_ref[...]
  ce_o[0] = ce[:, :HH]
  ce_o[1] = ce[:, HH:]


def _ce0(e0, cw, cb):
  BR = 2000
  return pl.pallas_call(
      _ce0_body,
      grid=(E // BR,),
      in_specs=[
          pl.BlockSpec((BR, H), lambda i: (i, 0)),
          pl.BlockSpec((H, H), lambda i: (0, 0)),
          pl.BlockSpec((1, H), lambda i: (0, 0)),
      ],
      out_specs=pl.BlockSpec((2, BR, HH), lambda i: (0, i, 0)),
      out_shape=jax.ShapeDtypeStruct((2, E, HH), jnp.float32),
  )(e0, cw, cb)


def _fc1_body(sf_ref, of_ref, w1a_ref, w1b_ref, b_ref, out_ref):
  out_ref[...] = (_dot(sf_ref[...], w1a_ref[...])
                  + _dot(of_ref[...], w1b_ref[...]) + b_ref[...])


def _fc1(feats, w1a, w1b, b1):
  BR = 2048
  g = T // BR
  return pl.pallas_call(
      _fc1_body,
      grid=(g,),
      in_specs=[
          pl.BlockSpec((BR, H), lambda i: (i, 0)),
          pl.BlockSpec((BR, H), lambda i: (i + g, 0)),
          pl.BlockSpec((H, FC), lambda i: (0, 0)),
          pl.BlockSpec((H, FC), lambda i: (0, 0)),
          pl.BlockSpec((1, FC), lambda i: (0, 0)),
      ],
      out_specs=pl.BlockSpec((BR, FC), lambda i: (i, 0)),
      out_shape=jax.ShapeDtypeStruct((T, FC), jnp.float32),
  )(feats, feats, w1a, w1b, b1)


def _x1stats_body(x_ref, out_ref):
  i = pl.program_id(0)

  @pl.when(i == 0)
  def _():
    out_ref[...] = jnp.zeros_like(out_ref)

  x = x_ref[...]
  out_ref[0:1, :] += jnp.sum(x, axis=0, keepdims=True)
  out_ref[1:2, :] += jnp.sum(x * x, axis=0, keepdims=True)


def _x1stats(x1):
  BR = 4096
  return pl.pallas_call(
      _x1stats_body,
      grid=(T // BR,),
      in_specs=[pl.BlockSpec((BR, FC), lambda i: (i, 0))],
      out_specs=pl.BlockSpec((8, FC), lambda i: (0, 0)),
      out_shape=jax.ShapeDtypeStruct((8, FC), jnp.float32),
  )(x1)


def _mlpout_body(x_ref, st_ref, g_ref, b_ref, w_ref, ob_ref, out_ref):
  m = st_ref[0:1, :] / T
  v = st_ref[1:2, :] / T - m * m
  yn = (x_ref[...] - m) / jnp.sqrt(v + 1e-5) * g_ref[...] + b_ref[...]
  yn = jnp.maximum(yn, 0.0)
  out_ref[...] = _dot(yn, w_ref[...]) + ob_ref[...]


def _mlpout(x1, st, g, b, w, ob):
  BR = 2048
  return pl.pallas_call(
      _mlpout_body,
      grid=(T // BR,),
      in_specs=[
          pl.BlockSpec((BR, FC), lambda i: (i, 0)),
          pl.BlockSpec((8, FC), lambda i: (0, 0)),
          pl.BlockSpec((1, FC), lambda i: (0, 0)),
          pl.BlockSpec((1, FC), lambda i: (0, 0)),
          pl.BlockSpec((FC, OD), lambda i: (0, 0)),
          pl.BlockSpec((1, OD), lambda i: (0, 0)),
      ],
      out_specs=pl.BlockSpec((BR, OD), lambda i: (i, 0)),
      out_shape=jax.ShapeDtypeStruct((T, OD), jnp.float32),
  )(x1, st, g, b, w, ob)


_gather_h0 = _make_gather(N, NP, 320)
_gather_e0 = _make_gather(OD, E, 400)
_gather_tf = _make_gather(NP, 2 * T, 512)
_edge_full = _make_edge(True)
_edge_last = _make_edge(False)


def kernel(node_feat, edge_feat, edge_index, norm_n, norm_e, triplets,
           h_emb, e_emb, A_w, B_w, C_w, D_w, E_w, A_b, B_b, C_b, D_b, E_b,
           bn_h_g, bn_h_b, bn_e_g, bn_e_b, fc1_w, fc1_b, bn1_g, bn1_b,
           out_w, out_b):
  i32 = jnp.int32
  src3 = edge_index[0].astype(i32).reshape(NS, NBLK, EB)
  dst3 = edge_index[1].astype(i32).reshape(NS, NBLK, EB)
  nf = jnp.concatenate([node_feat.astype(i32),
                        jnp.zeros((NP - N,), i32)])
  tf_idx = jnp.concatenate([triplets[:, 0].astype(i32),
                            triplets[:, 2].astype(i32)])
  nn = jnp.concatenate([norm_n, jnp.zeros((NP - N, 1), jnp.float32)])

  h = _gather_h0(h_emb, nf)
  e = _gather_e0(e_emb, edge_feat.astype(i32))

  r1 = lambda x: x.reshape(1, -1)

  for l in range(L):
    h_in = h
    ah, db2, eh2 = _node_mm(
        h, A_w[l], B_w[l], D_w[l], E_w[l],
        r1(A_b[l]), r1(B_b[l]), r1(D_b[l]), r1(E_b[l]))
    if l == 0:
      ce2 = _ce0(e, C_w[0], r1(C_b[0]))
    else:
      st = _estats(eij2, norm_e)
      res = _eupdate_ce(eij2, e, norm_e, st, r1(bn_e_g[l - 1]),
                        r1(bn_e_b[l - 1]), C_w[l], r1(C_b[l]),
                        write_e=(l < L - 1))
      if l < L - 1:
        ce2, e = res
      else:
        (ce2,) = res

    if l < L - 1:
      nd, eij2 = _edge_full(src3, dst3, db2, eh2, ce2)
    else:
      (nd,) = _edge_last(src3, dst3, db2, eh2, ce2)

    h = _h_update(ah, nd, nn, h_in, r1(bn_h_g[l]), r1(bn_h_b[l]))

  feats = _gather_tf(h, tf_idx)
  x1 = _fc1(feats, fc1_w[:H], fc1_w[H:], r1(fc1_b))
  st1 = _x1stats(x1)
  out = _mlpout(x1, st1, r1(bn1_g), r1(bn1_b), out_w, r1(out_b))
  return out


# speedup vs baseline: 1.0896x; 1.0896x over previous
import functools
import jax
import jax.numpy as jnp
from jax import lax
from jax.experimental import pallas as pl
from jax.experimental.pallas import tpu as pltpu
from jax.experimental.pallas import tpu_sc as plsc

N = 10000
NP = 10240
E = 320000
H = 128
HH = 64
L = 3
T = 32768
FC = 1000
OD = 474

NC, NS, LN = 2, 16, 16
NW = NC * NS

_SC_PARAMS = pltpu.CompilerParams(use_tc_tiling_on_sc=False)
_mesh = plsc.VectorSubcoreMesh(core_axis_name="c", subcore_axis_name="s")


def _make_gather(V, B, G):
  assert B % NW == 0
  b_per_w = B // NW
  assert b_per_w % G == 0 and G % 8 == 0

  @functools.partial(
      pl.kernel, mesh=_mesh, compiler_params=_SC_PARAMS,
      out_type=jax.ShapeDtypeStruct((B, H), jnp.float32),
      scratch_types=[
          pltpu.VMEM((G,), jnp.int32),
          pltpu.VMEM((G, H), jnp.float32),
          pltpu.SemaphoreType.DMA,
      ],
  )
  def k(table_h, idx_h, out_h, idx_v, rows_v, sem):
    wid = lax.axis_index("s") * NC + lax.axis_index("c")

    @pl.loop(0, b_per_w, step=G)
    def _(i):
      base = wid * b_per_w + i
      pltpu.sync_copy(idx_h.at[pl.ds(base, G)], idx_v)
      pltpu.async_copy(table_h.at[idx_v], rows_v, sem).wait()
      pltpu.sync_copy(rows_v, out_h.at[pl.ds(base, G)])

  return k


EB = 80
E_PER_S = E // NS
NBLK = E_PER_S // EB
ROWS_PER_S = NP // NS


def _make_edge(write_eij):
  outs = [jax.ShapeDtypeStruct((2, NP, H), jnp.float32)]
  if write_eij:
    outs.append(jax.ShapeDtypeStruct((2, E, HH), jnp.float32))

  @functools.partial(
      pl.kernel, mesh=_mesh, compiler_params=_SC_PARAMS,
      out_type=outs,
      scratch_types=[
          pltpu.VMEM((2, EB), jnp.int32),
          pltpu.VMEM((2, EB), jnp.int32),
          pltpu.VMEM((EB, H), jnp.float32),
          pltpu.VMEM((EB, HH), jnp.float32),
          pltpu.VMEM((EB, H), jnp.float32),
          pltpu.VMEM((EB, HH), jnp.float32),
          pltpu.VMEM((EB, HH), jnp.float32),
          pltpu.VMEM((EB, H), jnp.float32),
          pltpu.VMEM_SHARED((NP, H), jnp.float32),
          pltpu.SemaphoreType.DMA,
          pltpu.SemaphoreType.DMA,
          pltpu.SemaphoreType.DMA,
      ],
  )
  def k(src_h, dst_h, db_h, eh_h, ce_h, *rest):
    if write_eij:
      nd_h, eij_h = rest[:2]
      scratch = rest[2:]
    else:
      nd_h = rest[0]
      scratch = rest[1:]
    (idxa, idxb, gdba, g2a, gdbb, g2b, cev, scomb, acc,
     sem_a, sem_b, sem_c) = scratch
    c = lax.axis_index("c")
    sid = lax.axis_index("s")

    @pl.loop(0, EB)
    def _(r):
      @pl.loop(0, H, step=LN)
      def _(j):
        scomb[r, pl.ds(j, LN)] = jnp.zeros((LN,), jnp.float32)

    @pl.loop(0, ROWS_PER_S, step=EB)
    def _(r):
      pltpu.sync_copy(scomb, acc.at[pl.ds(sid * ROWS_PER_S + r, EB)])

    plsc.subcore_barrier()

    def fetch_idx(blk, idx2):
      pltpu.sync_copy(src_h.at[sid, blk], idx2.at[0])
      pltpu.sync_copy(dst_h.at[sid, blk], idx2.at[1])

    def issue(idx2, gdb, g2, sem):
      pltpu.async_copy(db_h.at[c].at[idx2.at[0]], gdb, sem)
      pltpu.async_copy(eh_h.at[c].at[idx2.at[1]], g2, sem)

    def wait_set(idx2, gdb, g2, sem):
      pltpu.make_async_copy(db_h.at[c].at[idx2.at[0]], gdb, sem).wait()
      pltpu.make_async_copy(eh_h.at[c].at[idx2.at[1]], g2, sem).wait()

    def issue_ce(blk):
      pltpu.async_copy(
          ce_h.at[c, pl.ds(sid * E_PER_S + blk * EB, EB)], cev, sem_c)

    def wait_ce(blk):
      pltpu.make_async_copy(
          ce_h.at[c, pl.ds(sid * E_PER_S + blk * EB, EB)], cev,
          sem_c).wait()

    def phase(blk, idx2, gdb, g2, sem):
      wait_ce(blk)
      wait_set(idx2, gdb, g2, sem)

      @pl.loop(0, EB)
      def _(r):
        for jj in range(0, HH, LN):
          j = pl.ds(jj, LN)
          eij = gdb[r, j] + g2[r, j] + cev[r, j]
          sg = 1.0 / (1.0 + jnp.exp(-eij))
          sgb = sg * gdb[r, pl.ds(HH + jj, LN)]
          cev[r, j] = eij
          scomb[r, pl.ds(HH + jj, LN)] = sg
          scomb[r, j] = sgb

      if write_eij:
        pltpu.sync_copy(
            cev, eij_h.at[c, pl.ds(sid * E_PER_S + blk * EB, EB)])
      pltpu.sync_copy(scomb, acc.at[idx2.at[1]], add=True)

      @pl.when(blk + 1 < NBLK)
      def _():
        issue_ce(blk + 1)

      @pl.when(blk + 2 < NBLK)
      def _():
        fetch_idx(blk + 2, idx2)
        issue(idx2, gdb, g2, sem)

    fetch_idx(0, idxa)
    issue(idxa, gdba, g2a, sem_a)
    fetch_idx(1, idxb)
    issue(idxb, gdbb, g2b, sem_b)
    issue_ce(0)

    @pl.loop(0, NBLK, step=2)
    def _(i):
      phase(i, idxa, gdba, g2a, sem_a)
      phase(i + 1, idxb, gdbb, g2b, sem_b)

    plsc.subcore_barrier()
    rb = sid * ROWS_PER_S
    pltpu.sync_copy(acc.at[pl.ds(rb, ROWS_PER_S)],
                    nd_h.at[c, pl.ds(rb, ROWS_PER_S)])

  return k


def _dot(a, b):
  return lax.dot_general(a, b, (((1,), (0,)), ((), ())),
                         preferred_element_type=jnp.float32)


def _node_mm_body(h_ref, aw, bw, dw, ew, ab, bb, db, eb,
                  ah_o, db_o, eh_o):
  h = h_ref[...]
  ah_o[...] = _dot(h, aw[...]) + ab[...]
  dh = _dot(h, dw[...]) + db[...]
  eh = _dot(h, ew[...]) + eb[...]
  bh = _dot(h, bw[...]) + bb[...]
  db_o[0] = jnp.concatenate([dh[:, :HH], bh[:, :HH]], axis=1)
  db_o[1] = jnp.concatenate([dh[:, HH:], bh[:, HH:]], axis=1)
  eh_o[0] = eh[:, :HH]
  eh_o[1] = eh[:, HH:]


def _node_mm(h, aw, bw, dw, ew, ab, bb, db, eb):
  BR = 2560
  g = NP // BR
  wspec = pl.BlockSpec((H, H), lambda i: (0, 0))
  bspec = pl.BlockSpec((1, H), lambda i: (0, 0))
  return pl.pallas_call(
      _node_mm_body,
      grid=(g,),
      in_specs=[pl.BlockSpec((BR, H), lambda i: (i, 0))] + [wspec] * 4
      + [bspec] * 4,
      out_specs=[
          pl.BlockSpec((BR, H), lambda i: (i, 0)),
          pl.BlockSpec((2, BR, H), lambda i: (0, i, 0)),
          pl.BlockSpec((2, BR, HH), lambda i: (0, i, 0)),
      ],
      out_shape=[
          jax.ShapeDtypeStruct((NP, H), jnp.float32),
          jax.ShapeDtypeStruct((2, NP, H), jnp.float32),
          jax.ShapeDtypeStruct((2, NP, HH), jnp.float32),
      ],
  )(h, aw, bw, dw, ew, ab, bb, db, eb)


def _h_update_body(ah_ref, nd_ref, nn_ref, hin_ref, g_ref, b_ref,
                   out_ref):
  num = jnp.concatenate([nd_ref[0][:, :HH], nd_ref[1][:, :HH]], axis=1)
  den = jnp.concatenate([nd_ref[0][:, HH:], nd_ref[1][:, HH:]], axis=1)
  t = (ah_ref[...] + num / (den + 1e-6)) * nn_ref[...]
  rid = lax.broadcasted_iota(jnp.int32, (NP, H), 0)
  mask = rid < N
  tm = jnp.where(mask, t, 0.0)
  m = jnp.sum(tm, axis=0, keepdims=True) / N
  v = jnp.sum(tm * tm, axis=0, keepdims=True) / N - m * m
  y = (t - m) / jnp.sqrt(v + 1e-5) * g_ref[...] + b_ref[...]
  out_ref[...] = hin_ref[...] + jnp.maximum(y, 0.0)


def _h_update(ah, nd, nn, hin, g, b):
  return pl.pallas_call(
      _h_update_body,
      grid=(1,),
      in_specs=[
          pl.BlockSpec((NP, H), lambda i: (0, 0)),
          pl.BlockSpec((2, NP, H), lambda i: (0, 0, 0)),
          pl.BlockSpec((NP, 1), lambda i: (0, 0)),
          pl.BlockSpec((NP, H), lambda i: (0, 0)),
          pl.BlockSpec((1, H), lambda i: (0, 0)),
          pl.BlockSpec((1, H), lambda i: (0, 0)),
      ],
      out_specs=pl.BlockSpec((NP, H), lambda i: (0, 0)),
      out_shape=jax.ShapeDtypeStruct((NP, H), jnp.float32),
  )(ah, nd, nn, hin, g, b)


def _estats_body(eij_ref, ne_ref, out_ref):
  i = pl.program_id(0)

  @pl.when(i == 0)
  def _():
    out_ref[...] = jnp.zeros_like(out_ref)

  y = jnp.concatenate([eij_ref[0], eij_ref[1]], axis=1) * ne_ref[...]
  s = jnp.sum(y, axis=0, keepdims=True)
  s2 = jnp.sum(y * y, axis=0, keepdims=True)
  out_ref[0:1, :] += s
  out_ref[1:2, :] += s2


def _estats(eij, ne):
  BR = 8000
  g = E // BR
  return pl.pallas_call(
      _estats_body,
      grid=(g,),
      in_specs=[
          pl.BlockSpec((2, BR, HH), lambda i: (0, i, 0)),
          pl.BlockSpec((BR, 1), lambda i: (i, 0)),
      ],
      out_specs=pl.BlockSpec((8, H), lambda i: (0, 0)),
      out_shape=jax.ShapeDtypeStruct((8, H), jnp.float32),
  )(eij, ne)


def _eupdate_ce_body(write_e, eij_ref, ein_ref, ne_ref, st_ref, g_ref, b_ref,
                     cw_ref, cb_ref, *outs):
  y = jnp.concatenate([eij_ref[0], eij_ref[1]], axis=1) * ne_ref[...]
  m = st_ref[0:1, :] / E
  v = st_ref[1:2, :] / E - m * m
  yn = (y - m) / jnp.sqrt(v + 1e-5) * g_ref[...] + b_ref[...]
  e_new = ein_ref[...] + jnp.maximum(yn, 0.0)
  ce = _dot(e_new, cw_ref[...]) + cb_ref[...]
  if write_e:
    ce_o, e_o = outs
    e_o[...] = e_new
  else:
    (ce_o,) = outs
  ce_o[0] = ce[:, :HH]
  ce_o[1] = ce[:, HH:]


def _eupdate_ce(eij, ein, ne, st, g, b, cw, cb, write_e):
  BR = 2000
  grid = E // BR
  out_specs = [pl.BlockSpec((2, BR, HH), lambda i: (0, i, 0))]
  out_shape = [jax.ShapeDtypeStruct((2, E, HH), jnp.float32)]
  if write_e:
    out_specs.append(pl.BlockSpec((BR, H), lambda i: (i, 0)))
    out_shape.append(jax.ShapeDtypeStruct((E, H), jnp.float32))
  return pl.pallas_call(
      functools.partial(_eupdate_ce_body, write_e),
      grid=(grid,),
      in_specs=[
          pl.BlockSpec((2, BR, HH), lambda i: (0, i, 0)),
          pl.BlockSpec((BR, H), lambda i: (i, 0)),
          pl.BlockSpec((BR, 1), lambda i: (i, 0)),
          pl.BlockSpec((8, H), lambda i: (0, 0)),
          pl.BlockSpec((1, H), lambda i: (0, 0)),
          pl.BlockSpec((1, H), lambda i: (0, 0)),
          pl.BlockSpec((H, H), lambda i: (0, 0)),
          pl.BlockSpec((1, H), lambda i: (0, 0)),
      ],
      out_specs=out_specs,
      out_shape=out_shape,
  )(eij, ein, ne, st, g, b, cw, cb)


def _ce0_body(e_ref, cw_ref, cb_ref, ce_o):
  ce = _dot(e_ref[...], cw_ref[...]) + cb_ref[...]
  ce_o[0] = ce[:, :HH]
  ce_o[1] = ce[:, HH:]


def _ce0(e0, cw, cb):
  BR = 2000
  return pl.pallas_call(
      _ce0_body,
      grid=(E // BR,),
      in_specs=[
          pl.BlockSpec((BR, H), lambda i: (i, 0)),
          pl.BlockSpec((H, H), lambda i: (0, 0)),
          pl.BlockSpec((1, H), lambda i: (0, 0)),
      ],
      out_specs=pl.BlockSpec((2, BR, HH), lambda i: (0, i, 0)),
      out_shape=jax.ShapeDtypeStruct((2, E, HH), jnp.float32),
  )(e0, cw, cb)


def _fc1_body(sf_ref, of_ref, w1a_ref, w1b_ref, b_ref, out_ref):
  out_ref[...] = (_dot(sf_ref[...], w1a_ref[...])
                  + _dot(of_ref[...], w1b_ref[...]) + b_ref[...])


def _fc1(feats, w1a, w1b, b1):
  BR = 2048
  g = T // BR
  return pl.pallas_call(
      _fc1_body,
      grid=(g,),
      in_specs=[
          pl.BlockSpec((BR, H), lambda i: (i, 0)),
          pl.BlockSpec((BR, H), lambda i: (i + g, 0)),
          pl.BlockSpec((H, FC), lambda i: (0, 0)),
          pl.BlockSpec((H, FC), lambda i: (0, 0)),
          pl.BlockSpec((1, FC), lambda i: (0, 0)),
      ],
      out_specs=pl.BlockSpec((BR, FC), lambda i: (i, 0)),
      out_shape=jax.ShapeDtypeStruct((T, FC), jnp.float32),
  )(feats, feats, w1a, w1b, b1)


def _x1stats_body(x_ref, out_ref):
  i = pl.program_id(0)

  @pl.when(i == 0)
  def _():
    out_ref[...] = jnp.zeros_like(out_ref)

  x = x_ref[...]
  out_ref[0:1, :] += jnp.sum(x, axis=0, keepdims=True)
  out_ref[1:2, :] += jnp.sum(x * x, axis=0, keepdims=True)


def _x1stats(x1):
  BR = 4096
  return pl.pallas_call(
      _x1stats_body,
      grid=(T // BR,),
      in_specs=[pl.BlockSpec((BR, FC), lambda i: (i, 0))],
      out_specs=pl.BlockSpec((8, FC), lambda i: (0, 0)),
      out_shape=jax.ShapeDtypeStruct((8, FC), jnp.float32),
  )(x1)


def _mlpout_body(x_ref, st_ref, g_ref, b_ref, w_ref, ob_ref, out_ref):
  m = st_ref[0:1, :] / T
  v = st_ref[1:2, :] / T - m * m
  yn = (x_ref[...] - m) / jnp.sqrt(v + 1e-5) * g_ref[...] + b_ref[...]
  yn = jnp.maximum(yn, 0.0)
  out_ref[...] = _dot(yn, w_ref[...]) + ob_ref[...]


def _mlpout(x1, st, g, b, w, ob):
  BR = 2048
  return pl.pallas_call(
      _mlpout_body,
      grid=(T // BR,),
      in_specs=[
          pl.BlockSpec((BR, FC), lambda i: (i, 0)),
          pl.BlockSpec((8, FC), lambda i: (0, 0)),
          pl.BlockSpec((1, FC), lambda i: (0, 0)),
          pl.BlockSpec((1, FC), lambda i: (0, 0)),
          pl.BlockSpec((FC, OD), lambda i: (0, 0)),
          pl.BlockSpec((1, OD), lambda i: (0, 0)),
      ],
      out_specs=pl.BlockSpec((BR, OD), lambda i: (i, 0)),
      out_shape=jax.ShapeDtypeStruct((T, OD), jnp.float32),
  )(x1, st, g, b, w, ob)


_gather_h0 = _make_gather(N, NP, 320)
_gather_e0 = _make_gather(OD, E, 400)
_gather_tf = _make_gather(NP, 2 * T, 512)
_edge_full = _make_edge(True)
_edge_last = _make_edge(False)


def kernel(node_feat, edge_feat, edge_index, norm_n, norm_e, triplets,
           h_emb, e_emb, A_w, B_w, C_w, D_w, E_w, A_b, B_b, C_b, D_b, E_b,
           bn_h_g, bn_h_b, bn_e_g, bn_e_b, fc1_w, fc1_b, bn1_g, bn1_b,
           out_w, out_b):
  i32 = jnp.int32
  src3 = edge_index[0].astype(i32).reshape(NS, NBLK, EB)
  dst3 = edge_index[1].astype(i32).reshape(NS, NBLK, EB)
  nf = jnp.concatenate([node_feat.astype(i32),
                        jnp.zeros((NP - N,), i32)])
  tf_idx = jnp.concatenate([triplets[:, 0].astype(i32),
                            triplets[:, 2].astype(i32)])
  nn = jnp.concatenate([norm_n, jnp.zeros((NP - N, 1), jnp.float32)])

  h = _gather_h0(h_emb, nf)
  e = _gather_e0(e_emb, edge_feat.astype(i32))

  r1 = lambda x: x.reshape(1, -1)

  for l in range(L):
    h_in = h
    ah, db2, eh2 = _node_mm(
        h, A_w[l], B_w[l], D_w[l], E_w[l],
        r1(A_b[l]), r1(B_b[l]), r1(D_b[l]), r1(E_b[l]))
    if l == 0:
      ce2 = _ce0(e, C_w[0], r1(C_b[0]))
    else:
      st = _estats(eij2, norm_e)
      res = _eupdate_ce(eij2, e, norm_e, st, r1(bn_e_g[l - 1]),
                        r1(bn_e_b[l - 1]), C_w[l], r1(C_b[l]),
                        write_e=(l < L - 1))
      if l < L - 1:
        ce2, e = res
      else:
        (ce2,) = res

    if l < L - 1:
      nd, eij2 = _edge_full(src3, dst3, db2, eh2, ce2)
    else:
      (nd,) = _edge_last(src3, dst3, db2, eh2, ce2)

    h = _h_update(ah, nd, nn, h_in, r1(bn_h_g[l]), r1(bn_h_b[l]))

  feats = _gather_tf(h, tf_idx)
  x1 = _fc1(feats, fc1_w[:H], fc1_w[H:], r1(fc1_b))
  st1 = _x1stats(x1)
  out = _mlpout(x1, st1, r1(bn1_g), r1(bn1_b), out_w, r1(out_b))
  return out


# speedup vs baseline: 1.6698x; 1.5325x over previous
import functools
import jax
import jax.numpy as jnp
from jax import lax
from jax.experimental import pallas as pl
from jax.experimental.pallas import tpu as pltpu
from jax.experimental.pallas import tpu_sc as plsc

N = 10000
NP = 10240
E = 320000
H = 128
HH = 64
L = 3
T = 32768
FC = 1000
OD = 474

NC, NS, LN = 2, 16, 16
NW = NC * NS

_SC_PARAMS = pltpu.CompilerParams(use_tc_tiling_on_sc=False)
_mesh = plsc.VectorSubcoreMesh(core_axis_name="c", subcore_axis_name="s")

E_PER_S = E // NS
ROWS_PER_S = NP // NS

EBA = 200
NBLKA = E_PER_S // EBA
EBC = 80
NBLKC = E_PER_S // EBC


def _make_gather(B, G):
  assert B % NW == 0
  b_per_w = B // NW
  assert b_per_w % G == 0 and G % 8 == 0

  @functools.partial(
      pl.kernel, mesh=_mesh, compiler_params=_SC_PARAMS,
      out_type=jax.ShapeDtypeStruct((B, H), jnp.float32),
      scratch_types=[
          pltpu.VMEM((G,), jnp.int32),
          pltpu.VMEM((G, H), jnp.float32),
          pltpu.SemaphoreType.DMA,
      ],
  )
  def k(table_h, idx_h, out_h, idx_v, rows_v, sem):
    wid = lax.axis_index("s") * NC + lax.axis_index("c")

    @pl.loop(0, b_per_w, step=G)
    def _(i):
      base = wid * b_per_w + i
      pltpu.sync_copy(idx_h.at[pl.ds(base, G)], idx_v)
      pltpu.async_copy(table_h.at[idx_v], rows_v, sem).wait()
      pltpu.sync_copy(rows_v, out_h.at[pl.ds(base, G)])

  return k


def _make_edge_a():
  @functools.partial(
      pl.kernel, mesh=_mesh, compiler_params=_SC_PARAMS,
      out_type=jax.ShapeDtypeStruct((2, E, HH), jnp.float32),
      scratch_types=[
          pltpu.VMEM((2, EBA), jnp.int32),
          pltpu.VMEM((2, EBA), jnp.int32),
          pltpu.VMEM((EBA, HH), jnp.float32),
          pltpu.VMEM((EBA, HH), jnp.float32),
          pltpu.VMEM((EBA, HH), jnp.float32),
          pltpu.VMEM((EBA, HH), jnp.float32),
          pltpu.VMEM((EBA, HH), jnp.float32),
          pltpu.VMEM((EBA, HH), jnp.float32),
          pltpu.SemaphoreType.DMA,
          pltpu.SemaphoreType.DMA,
          pltpu.SemaphoreType.DMA,
      ],
  )
  def k(src_h, dst_h, dh_h, eh_h, ce_h, eij_h,
        idxa, idxb, gda, g2a, gdb, g2b, cev, eijv, sem_a, sem_b, sem_c):
    c = lax.axis_index("c")
    sid = lax.axis_index("s")

    def fetch_idx(blk, idx2):
      pltpu.sync_copy(src_h.at[sid, blk], idx2.at[0])
      pltpu.sync_copy(dst_h.at[sid, blk], idx2.at[1])

    def issue(idx2, gd, g2, sem):
      pltpu.async_copy(dh_h.at[c].at[idx2.at[0]], gd, sem)
      pltpu.async_copy(eh_h.at[c].at[idx2.at[1]], g2, sem)

    def wait_set(idx2, gd, g2, sem):
      pltpu.make_async_copy(dh_h.at[c].at[idx2.at[0]], gd, sem).wait()
      pltpu.make_async_copy(eh_h.at[c].at[idx2.at[1]], g2, sem).wait()

    def issue_ce(blk):
      pltpu.async_copy(
          ce_h.at[c, pl.ds(sid * E_PER_S + blk * EBA, EBA)], cev, sem_c)

    def wait_ce(blk):
      pltpu.make_async_copy(
          ce_h.at[c, pl.ds(sid * E_PER_S + blk * EBA, EBA)], cev,
          sem_c).wait()

    def phase(blk, idx2, gd, g2, sem):
      wait_ce(blk)
      wait_set(idx2, gd, g2, sem)

      @pl.loop(0, EBA)
      def _(r):
        for jj in range(0, HH, LN):
          j = pl.ds(jj, LN)
          eijv[r, j] = gd[r, j] + g2[r, j] + cev[r, j]

      @pl.when(blk + 1 < NBLKA)
      def _():
        issue_ce(blk + 1)

      pltpu.sync_copy(
          eijv, eij_h.at[c, pl.ds(sid * E_PER_S + blk * EBA, EBA)])

      @pl.when(blk + 2 < NBLKA)
      def _():
        fetch_idx(blk + 2, idx2)
        issue(idx2, gd, g2, sem)

    fetch_idx(0, idxa)
    issue(idxa, gda, g2a, sem_a)
    fetch_idx(1, idxb)
    issue(idxb, gdb, g2b, sem_b)
    issue_ce(0)

    @pl.loop(0, NBLKA, step=2)
    def _(i):
      phase(i, idxa, gda, g2a, sem_a)
      phase(i + 1, idxb, gdb, g2b, sem_b)

  return k


def _make_edge_c():
  @functools.partial(
      pl.kernel, mesh=_mesh, compiler_params=_SC_PARAMS,
      out_type=[
          jax.ShapeDtypeStruct((2, NP, HH), jnp.float32),
          jax.ShapeDtypeStruct((2, NP, HH), jnp.float32),
      ],
      scratch_types=[
          pltpu.VMEM((2, EBC), jnp.int32),
          pltpu.VMEM((2, EBC), jnp.int32),
          pltpu.VMEM((EBC, HH), jnp.float32),
          pltpu.VMEM((EBC, HH), jnp.float32),
          pltpu.VMEM((EBC, HH), jnp.float32),
          pltpu.VMEM((EBC, HH), jnp.float32),
          pltpu.VMEM((EBC, HH), jnp.float32),
          pltpu.VMEM_SHARED((NP, HH), jnp.float32),
          pltpu.VMEM_SHARED((NP, HH), jnp.float32),
          pltpu.SemaphoreType.DMA,
          pltpu.SemaphoreType.DMA,
      ],
  )
  def k(src_h, dst_h, bh_h, sig_h, num_h, den_h,
        idxa, idxb, sga, gba, sgb_, gbb, numc, acc_num, acc_den,
        sem_a, sem_b):
    c = lax.axis_index("c")
    sid = lax.axis_index("s")

    @pl.loop(0, EBC)
    def _(r):
      for jj in range(0, HH, LN):
        numc[r, pl.ds(jj, LN)] = jnp.zeros((LN,), jnp.float32)

    @pl.loop(0, ROWS_PER_S, step=EBC)
    def _(r):
      pltpu.sync_copy(numc, acc_num.at[pl.ds(sid * ROWS_PER_S + r, EBC)])
      pltpu.sync_copy(numc, acc_den.at[pl.ds(sid * ROWS_PER_S + r, EBC)])

    plsc.subcore_barrier()

    def fetch_idx(blk, idx2):
      pltpu.sync_copy(src_h.at[sid, blk], idx2.at[0])
      pltpu.sync_copy(dst_h.at[sid, blk], idx2.at[1])

    def issue(blk, idx2, sg, gb, sem):
      pltpu.async_copy(
          sig_h.at[c, pl.ds(sid * E_PER_S + blk * EBC, EBC)], sg, sem)
      pltpu.async_copy(bh_h.at[c].at[idx2.at[0]], gb, sem)

    def wait_set(blk, idx2, sg, gb, sem):
      pltpu.make_async_copy(
          sig_h.at[c, pl.ds(sid * E_PER_S + blk * EBC, EBC)], sg,
          sem).wait()
      pltpu.make_async_copy(bh_h.at[c].at[idx2.at[0]], gb, sem).wait()

    def phase(blk, idx2, sg, gb, sem):
      wait_set(blk, idx2, sg, gb, sem)

      @pl.loop(0, EBC)
      def _(r):
        for jj in range(0, HH, LN):
          j = pl.ds(jj, LN)
          numc[r, j] = sg[r, j] * gb[r, j]

      pltpu.sync_copy(numc, acc_num.at[idx2.at[1]], add=True)
      pltpu.sync_copy(sg, acc_den.at[idx2.at[1]], add=True)

      @pl.when(blk + 2 < NBLKC)
      def _():
        fetch_idx(blk + 2, idx2)
        issue(blk + 2, idx2, sg, gb, sem)

    fetch_idx(0, idxa)
    issue(0, idxa, sga, gba, sem_a)
    fetch_idx(1, idxb)
    issue(1, idxb, sgb_, gbb, sem_b)

    @pl.loop(0, NBLKC, step=2)
    def _(i):
      phase(i, idxa, sga, gba, sem_a)
      phase(i + 1, idxb, sgb_, gbb, sem_b)

    plsc.subcore_barrier()
    rb = sid * ROWS_PER_S
    pltpu.sync_copy(acc_num.at[pl.ds(rb, ROWS_PER_S)],
                    num_h.at[c, pl.ds(rb, ROWS_PER_S)])
    pltpu.sync_copy(acc_den.at[pl.ds(rb, ROWS_PER_S)],
                    den_h.at[c, pl.ds(rb, ROWS_PER_S)])

  return k


def _dot(a, b):
  return lax.dot_general(a, b, (((1,), (0,)), ((), ())),
                         preferred_element_type=jnp.float32)


def _node_mm_body(h_ref, aw, bw, dw, ew, ab, bb, db, eb,
                  ah_o, dh_o, eh_o, bh_o):
  h = h_ref[...]
  ah_o[...] = _dot(h, aw[...]) + ab[...]
  dh = _dot(h, dw[...]) + db[...]
  eh = _dot(h, ew[...]) + eb[...]
  bh = _dot(h, bw[...]) + bb[...]
  dh_o[0] = dh[:, :HH]
  dh_o[1] = dh[:, HH:]
  eh_o[0] = eh[:, :HH]
  eh_o[1] = eh[:, HH:]
  bh_o[0] = bh[:, :HH]
  bh_o[1] = bh[:, HH:]


def _node_mm(h, aw, bw, dw, ew, ab, bb, db, eb):
  BR = 2560
  g = NP // BR
  wspec = pl.BlockSpec((H, H), lambda i: (0, 0))
  bspec = pl.BlockSpec((1, H), lambda i: (0, 0))
  hs = pl.BlockSpec((2, BR, HH), lambda i: (0, i, 0))
  hsh = jax.ShapeDtypeStruct((2, NP, HH), jnp.float32)
  return pl.pallas_call(
      _node_mm_body,
      grid=(g,),
      in_specs=[pl.BlockSpec((BR, H), lambda i: (i, 0))] + [wspec] * 4
      + [bspec] * 4,
      out_specs=[pl.BlockSpec((BR, H), lambda i: (i, 0)), hs, hs, hs],
      out_shape=[jax.ShapeDtypeStruct((NP, H), jnp.float32),
                 hsh, hsh, hsh],
  )(h, aw, bw, dw, ew, ab, bb, db, eb)


def _sig_body(with_stats, eij_ref, ne_ref, sig_o, st_o):
  lo = eij_ref[0]
  hi = eij_ref[1]
  sig_o[0] = jax.nn.sigmoid(lo)
  sig_o[1] = jax.nn.sigmoid(hi)
  if with_stats:
    i = pl.program_id(0)

    @pl.when(i == 0)
    def _():
      st_o[...] = jnp.zeros_like(st_o)

    y = jnp.concatenate([lo, hi], axis=1) * ne_ref[...]
    st_o[0:1, :] += jnp.sum(y, axis=0, keepdims=True)
    st_o[1:2, :] += jnp.sum(y * y, axis=0, keepdims=True)


def _sig(eij, ne, with_stats):
  BR = 4000
  return pl.pallas_call(
      functools.partial(_sig_body, with_stats),
      grid=(E // BR,),
      in_specs=[
          pl.BlockSpec((2, BR, HH), lambda i: (0, i, 0)),
          pl.BlockSpec((BR, 1), lambda i: (i, 0)),
      ],
      out_specs=[
          pl.BlockSpec((2, BR, HH), lambda i: (0, i, 0)),
          pl.BlockSpec((8, H), lambda i: (0, 0)),
      ],
      out_shape=[
          jax.ShapeDtypeStruct((2, E, HH), jnp.float32),
          jax.ShapeDtypeStruct((8, H), jnp.float32),
      ],
  )(eij, ne)


def _h_update_body(ah_ref, num_ref, den_ref, nn_ref, hin_ref, g_ref, b_ref,
                   out_ref):
  num = jnp.concatenate([num_ref[0], num_ref[1]], axis=1)
  den = jnp.concatenate([den_ref[0], den_ref[1]], axis=1)
  t = (ah_ref[...] + num / (den + 1e-6)) * nn_ref[...]
  rid = lax.broadcasted_iota(jnp.int32, (NP, H), 0)
  mask = rid < N
  tm = jnp.where(mask, t, 0.0)
  m = jnp.sum(tm, axis=0, keepdims=True) / N
  v = jnp.sum(tm * tm, axis=0, keepdims=True) / N - m * m
  y = (t - m) / jnp.sqrt(v + 1e-5) * g_ref[...] + b_ref[...]
  out_ref[...] = hin_ref[...] + jnp.maximum(y, 0.0)


def _h_update(ah, num, den, nn, hin, g, b):
  hs = pl.BlockSpec((2, NP, HH), lambda i: (0, 0, 0))
  return pl.pallas_call(
      _h_update_body,
      grid=(1,),
      in_specs=[
          pl.BlockSpec((NP, H), lambda i: (0, 0)), hs, hs,
          pl.BlockSpec((NP, 1), lambda i: (0, 0)),
          pl.BlockSpec((NP, H), lambda i: (0, 0)),
          pl.BlockSpec((1, H), lambda i: (0, 0)),
          pl.BlockSpec((1, H), lambda i: (0, 0)),
      ],
      out_specs=pl.BlockSpec((NP, H), lambda i: (0, 0)),
      out_shape=jax.ShapeDtypeStruct((NP, H), jnp.float32),
  )(ah, num, den, nn, hin, g, b)


def _eupdate_ce_body(write_e, eij_ref, ein_ref, ne_ref, st_ref, g_ref, b_ref,
                     cw_ref, cb_ref, *outs):
  y = jnp.concatenate([eij_ref[0], eij_ref[1]], axis=1) * ne_ref[...]
  m = st_ref[0:1, :] / E
  v = st_ref[1:2, :] / E - m * m
  yn = (y - m) / jnp.sqrt(v + 1e-5) * g_ref[...] + b_ref[...]
  e_new = ein_ref[...] + jnp.maximum(yn, 0.0)
  ce = _dot(e_new, cw_ref[...]) + cb_ref[...]
  if write_e:
    ce_o, e_o = outs
    e_o[...] = e_new
  else:
    (ce_o,) = outs
  ce_o[0] = ce[:, :HH]
  ce_o[1] = ce[:, HH:]


def _eupdate_ce(eij, ein, ne, st, g, b, cw, cb, write_e):
  BR = 2000
  grid = E // BR
  out_specs = [pl.BlockSpec((2, BR, HH), lambda i: (0, i, 0))]
  out_shape = [jax.ShapeDtypeStruct((2, E, HH), jnp.float32)]
  if write_e:
    out_specs.append(pl.BlockSpec((BR, H), lambda i: (i, 0)))
    out_shape.append(jax.ShapeDtypeStruct((E, H), jnp.float32))
  return pl.pallas_call(
      functools.partial(_eupdate_ce_body, write_e),
      grid=(grid,),
      in_specs=[
          pl.BlockSpec((2, BR, HH), lambda i: (0, i, 0)),
          pl.BlockSpec((BR, H), lambda i: (i, 0)),
          pl.BlockSpec((BR, 1), lambda i: (i, 0)),
          pl.BlockSpec((8, H), lambda i: (0, 0)),
          pl.BlockSpec((1, H), lambda i: (0, 0)),
          pl.BlockSpec((1, H), lambda i: (0, 0)),
          pl.BlockSpec((H, H), lambda i: (0, 0)),
          pl.BlockSpec((1, H), lambda i: (0, 0)),
      ],
      out_specs=out_specs,
      out_shape=out_shape,
  )(eij, ein, ne, st, g, b, cw, cb)


def _ce0_body(e_ref, cw_ref, cb_ref, ce_o):
  ce = _dot(e_ref[...], cw_ref[...]) + cb_ref[...]
  ce_o[0] = ce[:, :HH]
  ce_o[1] = ce[:, HH:]


def _ce0(e0, cw, cb):
  BR = 2000
  return pl.pallas_call(
      _ce0_body,
      grid=(E // BR,),
      in_specs=[
          pl.BlockSpec((BR, H), lambda i: (i, 0)),
          pl.BlockSpec((H, H), lambda i: (0, 0)),
          pl.BlockSpec((1, H), lambda i: (0, 0)),
      ],
      out_specs=pl.BlockSpec((2, BR, HH), lambda i: (0, i, 0)),
      out_shape=jax.ShapeDtypeStruct((2, E, HH), jnp.float32),
  )(e0, cw, cb)


def _fc1_body(sf_ref, of_ref, w1a_ref, w1b_ref, b_ref, out_ref):
  out_ref[...] = (_dot(sf_ref[...], w1a_ref[...])
                  + _dot(of_ref[...], w1b_ref[...]) + b_ref[...])


def _fc1(feats, w1a, w1b, b1):
  BR = 2048
  g = T // BR
  return pl.pallas_call(
      _fc1_body,
      grid=(g,),
      in_specs=[
          pl.BlockSpec((BR, H), lambda i: (i, 0)),
          pl.BlockSpec((BR, H), lambda i: (i + g, 0)),
          pl.BlockSpec((H, FC), lambda i: (0, 0)),
          pl.BlockSpec((H, FC), lambda i: (0, 0)),
          pl.BlockSpec((1, FC), lambda i: (0, 0)),
      ],
      out_specs=pl.BlockSpec((BR, FC), lambda i: (i, 0)),
      out_shape=jax.ShapeDtypeStruct((T, FC), jnp.float32),
  )(feats, feats, w1a, w1b, b1)


def _x1stats_body(x_ref, out_ref):
  i = pl.program_id(0)

  @pl.when(i == 0)
  def _():
    out_ref[...] = jnp.zeros_like(out_ref)

  x = x_ref[...]
  out_ref[0:1, :] += jnp.sum(x, axis=0, keepdims=True)
  out_ref[1:2, :] += jnp.sum(x * x, axis=0, keepdims=True)


def _x1stats(x1):
  BR = 4096
  return pl.pallas_call(
      _x1stats_body,
      grid=(T // BR,),
      in_specs=[pl.BlockSpec((BR, FC), lambda i: (i, 0))],
      out_specs=pl.BlockSpec((8, FC), lambda i: (0, 0)),
      out_shape=jax.ShapeDtypeStruct((8, FC), jnp.float32),
  )(x1)


def _mlpout_body(x_ref, st_ref, g_ref, b_ref, w_ref, ob_ref, out_ref):
  m = st_ref[0:1, :] / T
  v = st_ref[1:2, :] / T - m * m
  yn = (x_ref[...] - m) / jnp.sqrt(v + 1e-5) * g_ref[...] + b_ref[...]
  yn = jnp.maximum(yn, 0.0)
  out_ref[...] = _dot(yn, w_ref[...]) + ob_ref[...]


def _mlpout(x1, st, g, b, w, ob):
  BR = 2048
  return pl.pallas_call(
      _mlpout_body,
      grid=(T // BR,),
      in_specs=[
          pl.BlockSpec((BR, FC), lambda i: (i, 0)),
          pl.BlockSpec((8, FC), lambda i: (0, 0)),
          pl.BlockSpec((1, FC), lambda i: (0, 0)),
          pl.BlockSpec((1, FC), lambda i: (0, 0)),
          pl.BlockSpec((FC, OD), lambda i: (0, 0)),
          pl.BlockSpec((1, OD), lambda i: (0, 0)),
      ],
      out_specs=pl.BlockSpec((BR, OD), lambda i: (i, 0)),
      out_shape=jax.ShapeDtypeStruct((T, OD), jnp.float32),
  )(x1, st, g, b, w, ob)


_gather_h0 = _make_gather(NP, 320)
_gather_e0 = _make_gather(E, 400)
_gather_tf = _make_gather(2 * T, 512)
_edge_a = _make_edge_a()
_edge_c = _make_edge_c()


def kernel(node_feat, edge_feat, edge_index, norm_n, norm_e, triplets,
           h_emb, e_emb, A_w, B_w, C_w, D_w, E_w, A_b, B_b, C_b, D_b, E_b,
           bn_h_g, bn_h_b, bn_e_g, bn_e_b, fc1_w, fc1_b, bn1_g, bn1_b,
           out_w, out_b):
  i32 = jnp.int32
  srcA = edge_index[0].astype(i32).reshape(NS, NBLKA, EBA)
  dstA = edge_index[1].astype(i32).reshape(NS, NBLKA, EBA)
  srcC = edge_index[0].astype(i32).reshape(NS, NBLKC, EBC)
  dstC = edge_index[1].astype(i32).reshape(NS, NBLKC, EBC)
  nf = jnp.concatenate([node_feat.astype(i32),
                        jnp.zeros((NP - N,), i32)])
  tf_idx = jnp.concatenate([triplets[:, 0].astype(i32),
                            triplets[:, 2].astype(i32)])
  nn = jnp.concatenate([norm_n, jnp.zeros((NP - N, 1), jnp.float32)])

  h = _gather_h0(h_emb, nf)
  e = _gather_e0(e_emb, edge_feat.astype(i32))

  r1 = lambda x: x.reshape(1, -1)

  for l in range(L):
    h_in = h
    ah, dh2, eh2, bh2 = _node_mm(
        h, A_w[l], B_w[l], D_w[l], E_w[l],
        r1(A_b[l]), r1(B_b[l]), r1(D_b[l]), r1(E_b[l]))
    if l == 0:
      ce2 = _ce0(e, C_w[0], r1(C_b[0]))
    else:
      res = _eupdate_ce(eij2, e, norm_e, st, r1(bn_e_g[l - 1]),
                        r1(bn_e_b[l - 1]), C_w[l], r1(C_b[l]),
                        write_e=(l < L - 1))
      if l < L - 1:
        ce2, e = res
      else:
        (ce2,) = res

    eij2 = _edge_a(srcA, dstA, dh2, eh2, ce2)
    sig2, st = _sig(eij2, norm_e, with_stats=(l < L - 1))
    num2, den2 = _edge_c(srcC, dstC, bh2, sig2)

    h = _h_update(ah, num2, den2, nn, h_in, r1(bn_h_g[l]), r1(bn_h_b[l]))

  feats = _gather_tf(h, tf_idx)
  x1 = _fc1(feats, fc1_w[:H], fc1_w[H:], r1(fc1_b))
  st1 = _x1stats(x1)
  out = _mlpout(x1, st1, r1(bn1_g), r1(bn1_b), out_w, r1(out_b))
  return out


# speedup vs baseline: 1.6743x; 1.0027x over previous
import functools
import jax
import jax.numpy as jnp
from jax import lax
from jax.experimental import pallas as pl
from jax.experimental.pallas import tpu as pltpu
from jax.experimental.pallas import tpu_sc as plsc

N = 10000
NP = 10240
E = 320000
H = 128
HH = 64
L = 3
T = 32768
FC = 1000
OD = 474

NC, NS, LN = 2, 16, 16
NW = NC * NS

_SC_PARAMS = pltpu.CompilerParams(use_tc_tiling_on_sc=False)
_mesh = plsc.VectorSubcoreMesh(core_axis_name="c", subcore_axis_name="s")

E_PER_S = E // NS
ROWS_PER_S = NP // NS

EBA = 200
NBLKA = E_PER_S // EBA
EBC = 80
NBLKC = E_PER_S // EBC


def _make_gather(B, G):
  assert B % NW == 0
  b_per_w = B // NW
  assert b_per_w % G == 0 and G % 8 == 0

  @functools.partial(
      pl.kernel, mesh=_mesh, compiler_params=_SC_PARAMS,
      out_type=jax.ShapeDtypeStruct((B, H), jnp.float32),
      scratch_types=[
          pltpu.VMEM((G,), jnp.int32),
          pltpu.VMEM((G, H), jnp.float32),
          pltpu.SemaphoreType.DMA,
      ],
  )
  def k(table_h, idx_h, out_h, idx_v, rows_v, sem):
    wid = lax.axis_index("s") * NC + lax.axis_index("c")

    @pl.loop(0, b_per_w, step=G)
    def _(i):
      base = wid * b_per_w + i
      pltpu.sync_copy(idx_h.at[pl.ds(base, G)], idx_v)
      pltpu.async_copy(table_h.at[idx_v], rows_v, sem).wait()
      pltpu.sync_copy(rows_v, out_h.at[pl.ds(base, G)])

  return k


def _make_edge_a():
  @functools.partial(
      pl.kernel, mesh=_mesh, compiler_params=_SC_PARAMS,
      out_type=jax.ShapeDtypeStruct((2, E, HH), jnp.float32),
      scratch_types=[
          pltpu.VMEM((2, EBA), jnp.int32),
          pltpu.VMEM((2, EBA), jnp.int32),
          pltpu.VMEM((EBA, HH), jnp.float32),
          pltpu.VMEM((EBA, HH), jnp.float32),
          pltpu.VMEM((EBA, HH), jnp.float32),
          pltpu.VMEM((EBA, HH), jnp.float32),
          pltpu.VMEM((EBA, HH), jnp.float32),
          pltpu.SemaphoreType.DMA,
          pltpu.SemaphoreType.DMA,
      ],
  )
  def k(src_h, dst_h, dh_h, eh_h, gsum_h,
        idxa, idxb, gda, g2a, gdb, g2b, gsumv, sem_a, sem_b):
    c = lax.axis_index("c")
    sid = lax.axis_index("s")

    def fetch_idx(blk, idx2):
      pltpu.sync_copy(src_h.at[sid, blk], idx2.at[0])
      pltpu.sync_copy(dst_h.at[sid, blk], idx2.at[1])

    def issue(idx2, gd, g2, sem):
      pltpu.async_copy(dh_h.at[c].at[idx2.at[0]], gd, sem)
      pltpu.async_copy(eh_h.at[c].at[idx2.at[1]], g2, sem)

    def wait_set(idx2, gd, g2, sem):
      pltpu.make_async_copy(dh_h.at[c].at[idx2.at[0]], gd, sem).wait()
      pltpu.make_async_copy(eh_h.at[c].at[idx2.at[1]], g2, sem).wait()

    def phase(blk, idx2, gd, g2, sem):
      wait_set(idx2, gd, g2, sem)

      @pl.loop(0, EBA)
      def _(r):
        for jj in range(0, HH, LN):
          j = pl.ds(jj, LN)
          gsumv[r, j] = gd[r, j] + g2[r, j]

      pltpu.sync_copy(
          gsumv, gsum_h.at[c, pl.ds(sid * E_PER_S + blk * EBA, EBA)])

      @pl.when(blk + 2 < NBLKA)
      def _():
        fetch_idx(blk + 2, idx2)
        issue(idx2, gd, g2, sem)

    fetch_idx(0, idxa)
    issue(idxa, gda, g2a, sem_a)
    fetch_idx(1, idxb)
    issue(idxb, gdb, g2b, sem_b)

    @pl.loop(0, NBLKA, step=2)
    def _(i):
      phase(i, idxa, gda, g2a, sem_a)
      phase(i + 1, idxb, gdb, g2b, sem_b)

  return k


def _make_edge_c():
  @functools.partial(
      pl.kernel, mesh=_mesh, compiler_params=_SC_PARAMS,
      out_type=[
          jax.ShapeDtypeStruct((2, NP, HH), jnp.float32),
          jax.ShapeDtypeStruct((2, NP, HH), jnp.float32),
      ],
      scratch_types=[
          pltpu.VMEM((2, EBC), jnp.int32),
          pltpu.VMEM((2, EBC), jnp.int32),
          pltpu.VMEM((EBC, HH), jnp.float32),
          pltpu.VMEM((EBC, HH), jnp.float32),
          pltpu.VMEM((EBC, HH), jnp.float32),
          pltpu.VMEM((EBC, HH), jnp.float32),
          pltpu.VMEM((EBC, HH), jnp.float32),
          pltpu.VMEM_SHARED((NP, HH), jnp.float32),
          pltpu.VMEM_SHARED((NP, HH), jnp.float32),
          pltpu.SemaphoreType.DMA,
          pltpu.SemaphoreType.DMA,
      ],
  )
  def k(src_h, dst_h, bh_h, sig_h, num_h, den_h,
        idxa, idxb, sga, gba, sgb_, gbb, numc, acc_num, acc_den,
        sem_a, sem_b):
    c = lax.axis_index("c")
    sid = lax.axis_index("s")

    @pl.loop(0, EBC)
    def _(r):
      for jj in range(0, HH, LN):
        numc[r, pl.ds(jj, LN)] = jnp.zeros((LN,), jnp.float32)

    @pl.loop(0, ROWS_PER_S, step=EBC)
    def _(r):
      pltpu.sync_copy(numc, acc_num.at[pl.ds(sid * ROWS_PER_S + r, EBC)])
      pltpu.sync_copy(numc, acc_den.at[pl.ds(sid * ROWS_PER_S + r, EBC)])

    plsc.subcore_barrier()

    def fetch_idx(blk, idx2):
      pltpu.sync_copy(src_h.at[sid, blk], idx2.at[0])
      pltpu.sync_copy(dst_h.at[sid, blk], idx2.at[1])

    def issue(blk, idx2, sg, gb, sem):
      pltpu.async_copy(
          sig_h.at[c, pl.ds(sid * E_PER_S + blk * EBC, EBC)], sg, sem)
      pltpu.async_copy(bh_h.at[c].at[idx2.at[0]], gb, sem)

    def wait_set(blk, idx2, sg, gb, sem):
      pltpu.make_async_copy(
          sig_h.at[c, pl.ds(sid * E_PER_S + blk * EBC, EBC)], sg,
          sem).wait()
      pltpu.make_async_copy(bh_h.at[c].at[idx2.at[0]], gb, sem).wait()

    def phase(blk, idx2, sg, gb, sem):
      wait_set(blk, idx2, sg, gb, sem)

      @pl.loop(0, EBC)
      def _(r):
        for jj in range(0, HH, LN):
          j = pl.ds(jj, LN)
          numc[r, j] = sg[r, j] * gb[r, j]

      pltpu.sync_copy(numc, acc_num.at[idx2.at[1]], add=True)
      pltpu.sync_copy(sg, acc_den.at[idx2.at[1]], add=True)

      @pl.when(blk + 2 < NBLKC)
      def _():
        fetch_idx(blk + 2, idx2)
        issue(blk + 2, idx2, sg, gb, sem)

    fetch_idx(0, idxa)
    issue(0, idxa, sga, gba, sem_a)
    fetch_idx(1, idxb)
    issue(1, idxb, sgb_, gbb, sem_b)

    @pl.loop(0, NBLKC, step=2)
    def _(i):
      phase(i, idxa, sga, gba, sem_a)
      phase(i + 1, idxb, sgb_, gbb, sem_b)

    plsc.subcore_barrier()
    rb = sid * ROWS_PER_S
    pltpu.sync_copy(acc_num.at[pl.ds(rb, ROWS_PER_S)],
                    num_h.at[c, pl.ds(rb, ROWS_PER_S)])
    pltpu.sync_copy(acc_den.at[pl.ds(rb, ROWS_PER_S)],
                    den_h.at[c, pl.ds(rb, ROWS_PER_S)])

  return k


def _dot(a, b):
  return lax.dot_general(a, b, (((1,), (0,)), ((), ())),
                         preferred_element_type=jnp.float32)


def _node_mm_body(h_ref, aw, bw, dw, ew, ab, bb, db, eb,
                  ah_o, dh_o, eh_o, bh_o):
  h = h_ref[...]
  ah_o[...] = _dot(h, aw[...]) + ab[...]
  dh = _dot(h, dw[...]) + db[...]
  eh = _dot(h, ew[...]) + eb[...]
  bh = _dot(h, bw[...]) + bb[...]
  dh_o[0] = dh[:, :HH]
  dh_o[1] = dh[:, HH:]
  eh_o[0] = eh[:, :HH]
  eh_o[1] = eh[:, HH:]
  bh_o[0] = bh[:, :HH]
  bh_o[1] = bh[:, HH:]


def _node_mm(h, aw, bw, dw, ew, ab, bb, db, eb):
  BR = 2560
  g = NP // BR
  wspec = pl.BlockSpec((H, H), lambda i: (0, 0))
  bspec = pl.BlockSpec((1, H), lambda i: (0, 0))
  hs = pl.BlockSpec((2, BR, HH), lambda i: (0, i, 0))
  hsh = jax.ShapeDtypeStruct((2, NP, HH), jnp.float32)
  return pl.pallas_call(
      _node_mm_body,
      grid=(g,),
      in_specs=[pl.BlockSpec((BR, H), lambda i: (i, 0))] + [wspec] * 4
      + [bspec] * 4,
      out_specs=[pl.BlockSpec((BR, H), lambda i: (i, 0)), hs, hs, hs],
      out_shape=[jax.ShapeDtypeStruct((NP, H), jnp.float32),
                 hsh, hsh, hsh],
  )(h, aw, bw, dw, ew, ab, bb, db, eb)


def _sig_body(with_stats, gs_ref, ce_ref, ne_ref, sig_o, st_o):
  lo = gs_ref[0] + ce_ref[0]
  hi = gs_ref[1] + ce_ref[1]
  sig_o[0] = jax.nn.sigmoid(lo)
  sig_o[1] = jax.nn.sigmoid(hi)
  if with_stats:
    i = pl.program_id(0)

    @pl.when(i == 0)
    def _():
      st_o[...] = jnp.zeros_like(st_o)

    y = jnp.concatenate([lo, hi], axis=1) * ne_ref[...]
    st_o[0:1, :] += jnp.sum(y, axis=0, keepdims=True)
    st_o[1:2, :] += jnp.sum(y * y, axis=0, keepdims=True)


def _sig(gs, ce, ne, with_stats):
  BR = 4000
  return pl.pallas_call(
      functools.partial(_sig_body, with_stats),
      grid=(E // BR,),
      in_specs=[
          pl.BlockSpec((2, BR, HH), lambda i: (0, i, 0)),
          pl.BlockSpec((2, BR, HH), lambda i: (0, i, 0)),
          pl.BlockSpec((BR, 1), lambda i: (i, 0)),
      ],
      out_specs=[
          pl.BlockSpec((2, BR, HH), lambda i: (0, i, 0)),
          pl.BlockSpec((8, H), lambda i: (0, 0)),
      ],
      out_shape=[
          jax.ShapeDtypeStruct((2, E, HH), jnp.float32),
          jax.ShapeDtypeStruct((8, H), jnp.float32),
      ],
  )(gs, ce, ne)


def _h_update_body(ah_ref, num_ref, den_ref, nn_ref, hin_ref, g_ref, b_ref,
                   out_ref):
  num = jnp.concatenate([num_ref[0], num_ref[1]], axis=1)
  den = jnp.concatenate([den_ref[0], den_ref[1]], axis=1)
  t = (ah_ref[...] + num / (den + 1e-6)) * nn_ref[...]
  rid = lax.broadcasted_iota(jnp.int32, (NP, H), 0)
  mask = rid < N
  tm = jnp.where(mask, t, 0.0)
  m = jnp.sum(tm, axis=0, keepdims=True) / N
  v = jnp.sum(tm * tm, axis=0, keepdims=True) / N - m * m
  y = (t - m) / jnp.sqrt(v + 1e-5) * g_ref[...] + b_ref[...]
  out_ref[...] = hin_ref[...] + jnp.maximum(y, 0.0)


def _h_update(ah, num, den, nn, hin, g, b):
  hs = pl.BlockSpec((2, NP, HH), lambda i: (0, 0, 0))
  return pl.pallas_call(
      _h_update_body,
      grid=(1,),
      in_specs=[
          pl.BlockSpec((NP, H), lambda i: (0, 0)), hs, hs,
          pl.BlockSpec((NP, 1), lambda i: (0, 0)),
          pl.BlockSpec((NP, H), lambda i: (0, 0)),
          pl.BlockSpec((1, H), lambda i: (0, 0)),
          pl.BlockSpec((1, H), lambda i: (0, 0)),
      ],
      out_specs=pl.BlockSpec((NP, H), lambda i: (0, 0)),
      out_shape=jax.ShapeDtypeStruct((NP, H), jnp.float32),
  )(ah, num, den, nn, hin, g, b)


def _eupdate_ce_body(write_e, gs_ref, cep_ref, ein_ref, ne_ref, st_ref,
                     g_ref, b_ref, cw_ref, cb_ref, *outs):
  eij_lo = gs_ref[0] + cep_ref[0]
  eij_hi = gs_ref[1] + cep_ref[1]
  y = jnp.concatenate([eij_lo, eij_hi], axis=1) * ne_ref[...]
  m = st_ref[0:1, :] / E
  v = st_ref[1:2, :] / E - m * m
  yn = (y - m) / jnp.sqrt(v + 1e-5) * g_ref[...] + b_ref[...]
  e_new = ein_ref[...] + jnp.maximum(yn, 0.0)
  ce = _dot(e_new, cw_ref[...]) + cb_ref[...]
  if write_e:
    ce_o, e_o = outs
    e_o[...] = e_new
  else:
    (ce_o,) = outs
  ce_o[0] = ce[:, :HH]
  ce_o[1] = ce[:, HH:]


def _eupdate_ce(gs, cep, ein, ne, st, g, b, cw, cb, write_e):
  BR = 2000
  grid = E // BR
  out_specs = [pl.BlockSpec((2, BR, HH), lambda i: (0, i, 0))]
  out_shape = [jax.ShapeDtypeStruct((2, E, HH), jnp.float32)]
  if write_e:
    out_specs.append(pl.BlockSpec((BR, H), lambda i: (i, 0)))
    out_shape.append(jax.ShapeDtypeStruct((E, H), jnp.float32))
  return pl.pallas_call(
      functools.partial(_eupdate_ce_body, write_e),
      grid=(grid,),
      in_specs=[
          pl.BlockSpec((2, BR, HH), lambda i: (0, i, 0)),
          pl.BlockSpec((2, BR, HH), lambda i: (0, i, 0)),
          pl.BlockSpec((BR, H), lambda i: (i, 0)),
          pl.BlockSpec((BR, 1), lambda i: (i, 0)),
          pl.BlockSpec((8, H), lambda i: (0, 0)),
          pl.BlockSpec((1, H), lambda i: (0, 0)),
          pl.BlockSpec((1, H), lambda i: (0, 0)),
          pl.BlockSpec((H, H), lambda i: (0, 0)),
          pl.BlockSpec((1, H), lambda i: (0, 0)),
      ],
      out_specs=out_specs,
      out_shape=out_shape,
  )(gs, cep, ein, ne, st, g, b, cw, cb)


def _ce0_body(e_ref, cw_ref, cb_ref, ce_o):
  ce = _dot(e_ref[...], cw_ref[...]) + cb_ref[...]
  ce_o[0] = ce[:, :HH]
  ce_o[1] = ce[:, HH:]


def _ce0(e0, cw, cb):
  BR = 2000
  return pl.pallas_call(
      _ce0_body,
      grid=(E // BR,),
      in_specs=[
          pl.BlockSpec((BR, H), lambda i: (i, 0)),
          pl.BlockSpec((H, H), lambda i: (0, 0)),
          pl.BlockSpec((1, H), lambda i: (0, 0)),
      ],
      out_specs=pl.BlockSpec((2, BR, HH), lambda i: (0, i, 0)),
      out_shape=jax.ShapeDtypeStruct((2, E, HH), jnp.float32),
  )(e0, cw, cb)


def _fc1_body(sf_ref, of_ref, w1a_ref, w1b_ref, b_ref, out_ref):
  out_ref[...] = (_dot(sf_ref[...], w1a_ref[...])
                  + _dot(of_ref[...], w1b_ref[...]) + b_ref[...])


def _fc1(feats, w1a, w1b, b1):
  BR = 2048
  g = T // BR
  return pl.pallas_call(
      _fc1_body,
      grid=(g,),
      in_specs=[
          pl.BlockSpec((BR, H), lambda i: (i, 0)),
          pl.BlockSpec((BR, H), lambda i: (i + g, 0)),
          pl.BlockSpec((H, FC), lambda i: (0, 0)),
          pl.BlockSpec((H, FC), lambda i: (0, 0)),
          pl.BlockSpec((1, FC), lambda i: (0, 0)),
      ],
      out_specs=pl.BlockSpec((BR, FC), lambda i: (i, 0)),
      out_shape=jax.ShapeDtypeStruct((T, FC), jnp.float32),
  )(feats, feats, w1a, w1b, b1)


def _x1stats_body(x_ref, out_ref):
  i = pl.program_id(0)

  @pl.when(i == 0)
  def _():
    out_ref[...] = jnp.zeros_like(out_ref)

  x = x_ref[...]
  out_ref[0:1, :] += jnp.sum(x, axis=0, keepdims=True)
  out_ref[1:2, :] += jnp.sum(x * x, axis=0, keepdims=True)


def _x1stats(x1):
  BR = 4096
  return pl.pallas_call(
      _x1stats_body,
      grid=(T // BR,),
      in_specs=[pl.BlockSpec((BR, FC), lambda i: (i, 0))],
      out_specs=pl.BlockSpec((8, FC), lambda i: (0, 0)),
      out_shape=jax.ShapeDtypeStruct((8, FC), jnp.float32),
  )(x1)


def _mlpout_body(x_ref, st_ref, g_ref, b_ref, w_ref, ob_ref, out_ref):
  m = st_ref[0:1, :] / T
  v = st_ref[1:2, :] / T - m * m
  yn = (x_ref[...] - m) / jnp.sqrt(v + 1e-5) * g_ref[...] + b_ref[...]
  yn = jnp.maximum(yn, 0.0)
  out_ref[...] = _dot(yn, w_ref[...]) + ob_ref[...]


def _mlpout(x1, st, g, b, w, ob):
  BR = 2048
  return pl.pallas_call(
      _mlpout_body,
      grid=(T // BR,),
      in_specs=[
          pl.BlockSpec((BR, FC), lambda i: (i, 0)),
          pl.BlockSpec((8, FC), lambda i: (0, 0)),
          pl.BlockSpec((1, FC), lambda i: (0, 0)),
          pl.BlockSpec((1, FC), lambda i: (0, 0)),
          pl.BlockSpec((FC, OD), lambda i: (0, 0)),
          pl.BlockSpec((1, OD), lambda i: (0, 0)),
      ],
      out_specs=pl.BlockSpec((BR, OD), lambda i: (i, 0)),
      out_shape=jax.ShapeDtypeStruct((T, OD), jnp.float32),
  )(x1, st, g, b, w, ob)


_gather_h0 = _make_gather(NP, 320)
_gather_e0 = _make_gather(E, 400)
_gather_tf = _make_gather(2 * T, 512)
_edge_a = _make_edge_a()
_edge_c = _make_edge_c()


def kernel(node_feat, edge_feat, edge_index, norm_n, norm_e, triplets,
           h_emb, e_emb, A_w, B_w, C_w, D_w, E_w, A_b, B_b, C_b, D_b, E_b,
           bn_h_g, bn_h_b, bn_e_g, bn_e_b, fc1_w, fc1_b, bn1_g, bn1_b,
           out_w, out_b):
  i32 = jnp.int32
  srcA = edge_index[0].astype(i32).reshape(NS, NBLKA, EBA)
  dstA = edge_index[1].astype(i32).reshape(NS, NBLKA, EBA)
  srcC = edge_index[0].astype(i32).reshape(NS, NBLKC, EBC)
  dstC = edge_index[1].astype(i32).reshape(NS, NBLKC, EBC)
  nf = jnp.concatenate([node_feat.astype(i32),
                        jnp.zeros((NP - N,), i32)])
  tf_idx = jnp.concatenate([triplets[:, 0].astype(i32),
                            triplets[:, 2].astype(i32)])
  nn = jnp.concatenate([norm_n, jnp.zeros((NP - N, 1), jnp.float32)])

  h = _gather_h0(h_emb, nf)
  e = _gather_e0(e_emb, edge_feat.astype(i32))

  r1 = lambda x: x.reshape(1, -1)

  for l in range(L):
    h_in = h
    ah, dh2, eh2, bh2 = _node_mm(
        h, A_w[l], B_w[l], D_w[l], E_w[l],
        r1(A_b[l]), r1(B_b[l]), r1(D_b[l]), r1(E_b[l]))
    if l == 0:
      ce2 = _ce0(e, C_w[0], r1(C_b[0]))
    else:
      res = _eupdate_ce(gsum2, ce2, e, norm_e, st, r1(bn_e_g[l - 1]),
                        r1(bn_e_b[l - 1]), C_w[l], r1(C_b[l]),
                        write_e=(l < L - 1))
      if l < L - 1:
        ce2, e = res
      else:
        (ce2,) = res

    gsum2 = _edge_a(srcA, dstA, dh2, eh2)
    sig2, st = _sig(gsum2, ce2, norm_e, with_stats=(l < L - 1))
    num2, den2 = _edge_c(srcC, dstC, bh2, sig2)

    h = _h_update(ah, num2, den2, nn, h_in, r1(bn_h_g[l]), r1(bn_h_b[l]))

  feats = _gather_tf(h, tf_idx)
  x1 = _fc1(feats, fc1_w[:H], fc1_w[H:], r1(fc1_b))
  st1 = _x1stats(x1)
  out = _mlpout(x1, st1, r1(bn1_g), r1(bn1_b), out_w, r1(out_b))
  return out
